# baseline (speedup 1.0000x reference)
import jax
import jax.numpy as jnp
from jax import lax
from jax.experimental import pallas as pl
from jax.experimental.pallas import tpu as pltpu
from jax.experimental.pallas import tpu_sc as plsc

N_LEVELS = 16
N_FEAT = 2
LOG2_T = 19
TBL = 1 << LOG2_T
N_PTS = 524288
PRIME_I32 = -1640531535
MASK = (1 << LOG2_T) - 1

NC, NS = 2, 16
NW = NC * NS
PW = N_PTS // NW
C = 512
NCHUNK = PW // C
GH = C // 16
IDX_BLK = 128
RPC = C // IDX_BLK
PACK = 8
NBE = 3


def _body(xs_hbm, ys_hbm, tab_hbm, out_hbm,
          xsv, ysv, wxv0, wyv0, wxv1, wyv1, idxb0, idxb1, exid0, exid1,
          selv0, selv1, slot2v0, slot3v0, slot2v1, slot3v1,
          rows0, rows1, outt, sem0, sem1, semov):
    wid = lax.axis_index("s") * NC + lax.axis_index("c")
    base0 = wid * PW
    iota = lax.iota(jnp.int32, 16)
    ones16 = jnp.ones((16,), jnp.int32)
    prime = jnp.int32(PRIME_I32)
    mask = jnp.int32(MASK)
    wxv = (wxv0, wxv1)
    wyv = (wyv0, wyv1)
    idxb = (idxb0, idxb1)
    exid = (exid0, exid1)
    selv = (selv0, selv1)
    slot2v = (slot2v0, slot2v1)
    slot3v = (slot3v0, slot3v1)
    rows = (rows0, rows1)
    sems = (sem0, sem1)

    def zero_body(i, _):
        exid0[pl.ds(i * 16, 16)] = iota * 0
        exid1[pl.ds(i * 16, 16)] = iota * 0
        return 0

    lax.fori_loop(0, 2 * C // 16, zero_body, 0)

    def hash_level(li, b):
        res_f = jnp.float32(256.0 * (2.0 ** li))
        lrow = jnp.int32(li << (LOG2_T - 2))
        wxb, wyb, idb, exb = wxv[b], wyv[b], idxb[b], exid[b]
        selb, s2b, s3b = selv[b], slot2v[b], slot3v[b]

        def hash_body(g, cnt):
            off = g * 16
            ipoint = iota + off
            xs = xsv[pl.ds(off, 16)]
            ys = ysv[pl.ds(off, 16)]
            xi = xs * res_f
            yi = ys * res_f
            vx = xi.astype(jnp.int32)
            vy = yi.astype(jnp.int32)
            wxb[pl.ds(off, 16)] = xi - vx.astype(jnp.float32)
            wyb[pl.ds(off, 16)] = yi - vy.astype(jnp.float32)
            yp = vy * prime
            yp1 = yp + prime
            vx1 = vx + 1
            h0 = (vx ^ yp) & mask
            h1 = (vx ^ yp1) & mask
            h2 = (vx1 ^ yp) & mask
            h3 = (vx1 ^ yp1) & mask
            row = g >> 3
            col = (g & 7) * 16
            idb[row, pl.ds(col, 16)] = (h0 >> 2) + lrow
            idb[row + RPC, pl.ds(col, 16)] = (h1 >> 2) + lrow
            selb[pl.ds(off, 16)] = (h0 & 3) * 2
            selb[pl.ds(C + off, 16)] = (h1 & 3) * 2
            selb[pl.ds(2 * C + off, 16)] = (h2 & 3) * 2
            selb[pl.ds(3 * C + off, 16)] = (h3 & 3) * 2
            nb = (vx & 3) == 3
            inc = plsc.cumsum(ones16, mask=nb)
            p0 = 2 * cnt + 2 * inc - 2
            plsc.store_scatter(exb, [p0], (h2 >> 2) + lrow, mask=nb)
            plsc.store_scatter(exb, [p0 + 1], (h3 >> 2) + lrow, mask=nb)
            s2b[pl.ds(off, 16)] = jnp.where(nb, 2 * C + p0, ipoint)
            s3b[pl.ds(off, 16)] = jnp.where(nb, 2 * C + p0 + 1, C + ipoint)
            return cnt + plsc.all_reduce_population_count(nb)

        cnt = lax.fori_loop(0, GH, hash_body, jnp.zeros((16,), jnp.int32))
        return (2 * cnt[0] + (IDX_BLK - 1)) >> 7

    def fire_level(b):
        cps = [pltpu.async_copy(
            tab_hbm.at[idxb[b].at[r]],
            rows[b].at[pl.ds(r * IDX_BLK, IDX_BLK)],
            sems[b]) for r in range(2 * RPC)]
        cps += [pltpu.async_copy(
            tab_hbm.at[exid[b].at[pl.ds(e * IDX_BLK, IDX_BLK)]],
            rows[b].at[pl.ds(2 * C + e * IDX_BLK, IDX_BLK)],
            sems[b]) for e in range(NBE)]
        return cps

    def drain_overflow(b, nblk):
        def ov(r, _):
            pltpu.async_copy(
                tab_hbm.at[exid[b].at[pl.ds(r * IDX_BLK, IDX_BLK)]],
                rows[b].at[pl.ds(2 * C + r * IDX_BLK, IDX_BLK)],
                semov).wait()
            return 0

        lax.fori_loop(NBE, jnp.maximum(nblk, NBE), ov, 0)

    def bil_level(li, b):
        wxb, wyb, selb, rowsb = wxv[b], wyv[b], selv[b], rows[b]
        s2b, s3b = slot2v[b], slot3v[b]
        col0 = jnp.full((16,), 2 * li, jnp.int32)
        col1 = col0 + 1

        def bil_body(g, _):
            off = g * 16
            ipoint = iota + off
            wx = wxb[pl.ds(off, 16)]
            wy = wyb[pl.ds(off, 16)]
            s0 = selb[pl.ds(off, 16)]
            s1 = selb[pl.ds(C + off, 16)]
            s2 = selb[pl.ds(2 * C + off, 16)]
            s3 = selb[pl.ds(3 * C + off, 16)]
            t2 = s2b[pl.ds(off, 16)]
            t3 = s3b[pl.ds(off, 16)]
            omx = 1.0 - wx
            omy = 1.0 - wy
            for f, colv in ((0, col0), (1, col1)):
                r0 = plsc.load_gather(rowsb, [ipoint, s0 + f])
                r1 = plsc.load_gather(rowsb, [ipoint + C, s1 + f])
                r2 = plsc.load_gather(rowsb, [t2, s2 + f])
                r3 = plsc.load_gather(rowsb, [t3, s3 + f])
                c0 = r0 * omx + r2 * wx
                c1 = r1 * omx + r3 * wx
                plsc.store_scatter(outt, [ipoint, colv], c0 * omy + c1 * wy)
            return 0

        lax.fori_loop(0, GH, bil_body, 0)

    def chunk_body(ci, _):
        base = base0 + ci * C
        pltpu.sync_copy(xs_hbm.at[pl.ds(base, C)], xsv)
        pltpu.sync_copy(ys_hbm.at[pl.ds(base, C)], ysv)

        nblk = hash_level(0, 0)
        cps = fire_level(0)
        for li in range(N_LEVELS):
            b = li & 1
            nxt_cps = None
            nxt_nblk = None
            if li + 1 < N_LEVELS:
                nxt_nblk = hash_level(li + 1, 1 - b)
                nxt_cps = fire_level(1 - b)
            drain_overflow(b, nblk)
            for cp in cps:
                cp.wait()
            bil_level(li, b)
            cps = nxt_cps
            nblk = nxt_nblk

        pltpu.sync_copy(outt, out_hbm.at[pl.ds(base, C)])
        return 0

    lax.fori_loop(0, NCHUNK, chunk_body, 0)


def kernel(x, tables):
    xs = x[:, 0]
    ys = x[:, 1]
    tab = tables.reshape(N_LEVELS * TBL * N_FEAT // PACK, PACK)
    mesh = plsc.VectorSubcoreMesh(
        core_axis_name="c", subcore_axis_name="s",
        num_cores=NC, num_subcores=NS)
    f = pl.kernel(
        _body,
        out_type=jax.ShapeDtypeStruct((N_PTS, N_LEVELS * N_FEAT), jnp.float32),
        mesh=mesh,
        compiler_params=pltpu.CompilerParams(
            needs_layout_passes=False, use_tc_tiling_on_sc=False),
        scratch_types=[
            pltpu.VMEM((C,), jnp.float32),
            pltpu.VMEM((C,), jnp.float32),
            pltpu.VMEM((C,), jnp.float32),
            pltpu.VMEM((C,), jnp.float32),
            pltpu.VMEM((C,), jnp.float32),
            pltpu.VMEM((C,), jnp.float32),
            pltpu.VMEM((2 * RPC, IDX_BLK), jnp.int32),
            pltpu.VMEM((2 * RPC, IDX_BLK), jnp.int32),
            pltpu.VMEM((2 * C,), jnp.int32),
            pltpu.VMEM((2 * C,), jnp.int32),
            pltpu.VMEM((4 * C,), jnp.int32),
            pltpu.VMEM((4 * C,), jnp.int32),
            pltpu.VMEM((C,), jnp.int32),
            pltpu.VMEM((C,), jnp.int32),
            pltpu.VMEM((C,), jnp.int32),
            pltpu.VMEM((C,), jnp.int32),
            pltpu.VMEM((4 * C, PACK), jnp.float32),
            pltpu.VMEM((4 * C, PACK), jnp.float32),
            pltpu.VMEM((C, N_LEVELS * N_FEAT), jnp.float32),
            pltpu.SemaphoreType.DMA,
            pltpu.SemaphoreType.DMA,
            pltpu.SemaphoreType.DMA,
        ],
    )
    return f(xs, ys, tab)


# speedup vs baseline: 1.0005x; 1.0005x over previous
import jax
import jax.numpy as jnp
from jax import lax
from jax.experimental import pallas as pl
from jax.experimental.pallas import tpu as pltpu
from jax.experimental.pallas import tpu_sc as plsc

N_LEVELS = 16
N_FEAT = 2
LOG2_T = 19
TBL = 1 << LOG2_T
N_PTS = 524288
PRIME_I32 = -1640531535
MASK = (1 << LOG2_T) - 1

NC, NS = 2, 16
NW = NC * NS
PW = N_PTS // NW
C = 512
NCHUNK = PW // C
GH = C // 16
IDX_BLK = 128
RPC = C // IDX_BLK
PACK = 8
NBE = 3


def _body(xs_hbm, ys_hbm, tab_hbm, out_hbm,
          xsv, ysv, wxv0, wyv0, wxv1, wyv1, idxb0, idxb1, exid0, exid1,
          selv0, selv1, slot2v0, slot3v0, slot2v1, slot3v1,
          rows0, rows1, outt, sem0, sem1, semov):
    wid = lax.axis_index("s") * NC + lax.axis_index("c")
    base0 = wid * PW
    iota = lax.iota(jnp.int32, 16)
    ones16 = jnp.ones((16,), jnp.int32)
    prime = jnp.int32(PRIME_I32)
    mask = jnp.int32(MASK)
    wxv = (wxv0, wxv1)
    wyv = (wyv0, wyv1)
    idxb = (idxb0, idxb1)
    exid = (exid0, exid1)
    selv = (selv0, selv1)
    slot2v = (slot2v0, slot2v1)
    slot3v = (slot3v0, slot3v1)
    rows = (rows0, rows1)
    sems = (sem0, sem1)

    def zero_body(i, _):
        exid0[pl.ds(i * 16, 16)] = iota * 0
        exid1[pl.ds(i * 16, 16)] = iota * 0
        return 0

    lax.fori_loop(0, 2 * C // 16, zero_body, 0)

    def hash_level(li, b):
        res_f = jnp.float32(256.0 * (2.0 ** li))
        lrow = jnp.int32(li << (LOG2_T - 2))
        wxb, wyb, idb, exb = wxv[b], wyv[b], idxb[b], exid[b]
        selb, s2b, s3b = selv[b], slot2v[b], slot3v[b]

        def hash_body(g, cnt):
            off = g * 16
            ipoint = iota + off
            xs = xsv[pl.ds(off, 16)]
            ys = ysv[pl.ds(off, 16)]
            xi = xs * res_f
            yi = ys * res_f
            vx = xi.astype(jnp.int32)
            vy = yi.astype(jnp.int32)
            wxb[pl.ds(off, 16)] = xi - vx.astype(jnp.float32)
            wyb[pl.ds(off, 16)] = yi - vy.astype(jnp.float32)
            yp = vy * prime
            yp1 = yp + prime
            vx1 = vx + 1
            h0 = (vx ^ yp) & mask
            h1 = (vx ^ yp1) & mask
            h2 = (vx1 ^ yp) & mask
            h3 = (vx1 ^ yp1) & mask
            row = g >> 3
            col = (g & 7) * 16
            idb[row, pl.ds(col, 16)] = (h0 >> 2) + lrow
            idb[row + RPC, pl.ds(col, 16)] = (h1 >> 2) + lrow
            selb[pl.ds(off, 16)] = (h0 & 3) * 2
            selb[pl.ds(C + off, 16)] = (h1 & 3) * 2
            selb[pl.ds(2 * C + off, 16)] = (h2 & 3) * 2
            selb[pl.ds(3 * C + off, 16)] = (h3 & 3) * 2
            nb = (vx & 3) == 3
            inc = plsc.cumsum(ones16, mask=nb)
            p0 = 2 * cnt + 2 * inc - 2
            plsc.store_scatter(exb, [p0], (h2 >> 2) + lrow, mask=nb)
            plsc.store_scatter(exb, [p0 + 1], (h3 >> 2) + lrow, mask=nb)
            s2b[pl.ds(off, 16)] = jnp.where(nb, 2 * C + p0, ipoint)
            s3b[pl.ds(off, 16)] = jnp.where(nb, 2 * C + p0 + 1, C + ipoint)
            return cnt + plsc.all_reduce_population_count(nb)

        cnt = lax.fori_loop(0, GH, hash_body, jnp.zeros((16,), jnp.int32))
        return (2 * cnt[0] + (IDX_BLK - 1)) >> 7

    def fire_level(b):
        cps = [pltpu.async_copy(
            tab_hbm.at[idxb[b].at[r]],
            rows[b].at[pl.ds(r * IDX_BLK, IDX_BLK)],
            sems[b]) for r in range(2 * RPC)]
        cps += [pltpu.async_copy(
            tab_hbm.at[exid[b].at[pl.ds(e * IDX_BLK, IDX_BLK)]],
            rows[b].at[pl.ds(2 * C + e * IDX_BLK, IDX_BLK)],
            sems[b]) for e in range(NBE)]
        return cps

    def drain_overflow(b, nblk):
        def full_drain():
            cps = [pltpu.async_copy(
                tab_hbm.at[exid[b].at[pl.ds(r * IDX_BLK, IDX_BLK)]],
                rows[b].at[pl.ds(2 * C + r * IDX_BLK, IDX_BLK)],
                semov) for r in range(NBE, 2 * C // IDX_BLK)]
            for cp in cps:
                cp.wait()

        lax.cond(nblk > NBE, full_drain, lambda: None)

    def bil_level(li, b):
        wxb, wyb, selb, rowsb = wxv[b], wyv[b], selv[b], rows[b]
        s2b, s3b = slot2v[b], slot3v[b]
        col0 = jnp.full((16,), 2 * li, jnp.int32)
        col1 = col0 + 1

        def bil_body(g, _):
            off = g * 16
            ipoint = iota + off
            wx = wxb[pl.ds(off, 16)]
            wy = wyb[pl.ds(off, 16)]
            s0 = selb[pl.ds(off, 16)]
            s1 = selb[pl.ds(C + off, 16)]
            s2 = selb[pl.ds(2 * C + off, 16)]
            s3 = selb[pl.ds(3 * C + off, 16)]
            t2 = s2b[pl.ds(off, 16)]
            t3 = s3b[pl.ds(off, 16)]
            omx = 1.0 - wx
            omy = 1.0 - wy
            for f, colv in ((0, col0), (1, col1)):
                r0 = plsc.load_gather(rowsb, [ipoint, s0 + f])
                r1 = plsc.load_gather(rowsb, [ipoint + C, s1 + f])
                r2 = plsc.load_gather(rowsb, [t2, s2 + f])
                r3 = plsc.load_gather(rowsb, [t3, s3 + f])
                c0 = r0 * omx + r2 * wx
                c1 = r1 * omx + r3 * wx
                plsc.store_scatter(outt, [ipoint, colv], c0 * omy + c1 * wy)
            return 0

        lax.fori_loop(0, GH, bil_body, 0)

    def chunk_body(ci, _):
        base = base0 + ci * C
        pltpu.sync_copy(xs_hbm.at[pl.ds(base, C)], xsv)
        pltpu.sync_copy(ys_hbm.at[pl.ds(base, C)], ysv)

        nblk = hash_level(0, 0)
        cps = fire_level(0)
        for li in range(N_LEVELS):
            b = li & 1
            nxt_cps = None
            nxt_nblk = None
            if li + 1 < N_LEVELS:
                nxt_nblk = hash_level(li + 1, 1 - b)
                nxt_cps = fire_level(1 - b)
            drain_overflow(b, nblk)
            for cp in cps:
                cp.wait()
            bil_level(li, b)
            cps = nxt_cps
            nblk = nxt_nblk

        pltpu.sync_copy(outt, out_hbm.at[pl.ds(base, C)])
        return 0

    lax.fori_loop(0, NCHUNK, chunk_body, 0)


def kernel(x, tables):
    xs = x[:, 0]
    ys = x[:, 1]
    tab = tables.reshape(N_LEVELS * TBL * N_FEAT // PACK, PACK)
    mesh = plsc.VectorSubcoreMesh(
        core_axis_name="c", subcore_axis_name="s",
        num_cores=NC, num_subcores=NS)
    f = pl.kernel(
        _body,
        out_type=jax.ShapeDtypeStruct((N_PTS, N_LEVELS * N_FEAT), jnp.float32),
        mesh=mesh,
        compiler_params=pltpu.CompilerParams(
            needs_layout_passes=False, use_tc_tiling_on_sc=False),
        scratch_types=[
            pltpu.VMEM((C,), jnp.float32),
            pltpu.VMEM((C,), jnp.float32),
            pltpu.VMEM((C,), jnp.float32),
            pltpu.VMEM((C,), jnp.float32),
            pltpu.VMEM((C,), jnp.float32),
            pltpu.VMEM((C,), jnp.float32),
            pltpu.VMEM((2 * RPC, IDX_BLK), jnp.int32),
            pltpu.VMEM((2 * RPC, IDX_BLK), jnp.int32),
            pltpu.VMEM((2 * C,), jnp.int32),
            pltpu.VMEM((2 * C,), jnp.int32),
            pltpu.VMEM((4 * C,), jnp.int32),
            pltpu.VMEM((4 * C,), jnp.int32),
            pltpu.VMEM((C,), jnp.int32),
            pltpu.VMEM((C,), jnp.int32),
            pltpu.VMEM((C,), jnp.int32),
            pltpu.VMEM((C,), jnp.int32),
            pltpu.VMEM((4 * C, PACK), jnp.float32),
            pltpu.VMEM((4 * C, PACK), jnp.float32),
            pltpu.VMEM((C, N_LEVELS * N_FEAT), jnp.float32),
            pltpu.SemaphoreType.DMA,
            pltpu.SemaphoreType.DMA,
            pltpu.SemaphoreType.DMA,
        ],
    )
    return f(xs, ys, tab)


# speedup vs baseline: 1.4003x; 1.3996x over previous
import jax
import jax.numpy as jnp
from jax import lax
from jax.experimental import pallas as pl
from jax.experimental.pallas import tpu as pltpu
from jax.experimental.pallas import tpu_sc as plsc

N_LEVELS = 16
N_FEAT = 2
LOG2_T = 19
TBL = 1 << LOG2_T
N_PTS = 524288
PRIME_I32 = -1640531535
MASK = (1 << LOG2_T) - 1

NC, NS = 2, 16
NW = NC * NS
PW = N_PTS // NW
C = 512
NCHUNK = PW // C
GH = C // 16
IDX_BLK = 128
RPC = C // IDX_BLK
PACK = 8
UNROLL = 4


def _body(xs_hbm, ys_hbm, tab_hbm, out_hbm,
          xsv, ysv, wxv0, wyv0, wxv1, wyv1, idxv0, selv0, idxv1, selv1,
          rows0, rows1, outt, sem0, sem1):
    wid = lax.axis_index("s") * NC + lax.axis_index("c")
    base0 = wid * PW
    iota = lax.iota(jnp.int32, 16)
    prime = jnp.int32(PRIME_I32)
    mask = jnp.int32(MASK)
    wxv = (wxv0, wxv1)
    wyv = (wyv0, wyv1)
    idxv = (idxv0, idxv1)
    selv = (selv0, selv1)
    rows = (rows0, rows1)
    sems = (sem0, sem1)

    def hash_level(li, b):
        res_f = jnp.float32(256.0 * (2.0 ** li))
        lrow = jnp.int32(li << (LOG2_T - 2))
        wxb, wyb, idxb, selb = wxv[b], wyv[b], idxv[b], selv[b]

        def hash_one(g):
            off = g * 16
            xs = xsv[pl.ds(off, 16)]
            ys = ysv[pl.ds(off, 16)]
            xi = xs * res_f
            yi = ys * res_f
            vx = xi.astype(jnp.int32)
            vy = yi.astype(jnp.int32)
            wxb[pl.ds(off, 16)] = xi - vx.astype(jnp.float32)
            wyb[pl.ds(off, 16)] = yi - vy.astype(jnp.float32)
            yp = vy * prime
            yp1 = yp + prime
            vx1 = vx + 1
            h0 = (vx ^ yp) & mask
            h1 = (vx ^ yp1) & mask
            h2 = (vx1 ^ yp) & mask
            h3 = (vx1 ^ yp1) & mask
            row = g >> 3
            col = (g & 7) * 16
            idxb[row, pl.ds(col, 16)] = (h0 >> 2) + lrow
            idxb[row + RPC, pl.ds(col, 16)] = (h1 >> 2) + lrow
            idxb[row + 2 * RPC, pl.ds(col, 16)] = (h2 >> 2) + lrow
            idxb[row + 3 * RPC, pl.ds(col, 16)] = (h3 >> 2) + lrow
            selb[pl.ds(off, 16)] = (h0 & 3) * 2
            selb[pl.ds(C + off, 16)] = (h1 & 3) * 2
            selb[pl.ds(2 * C + off, 16)] = (h2 & 3) * 2
            selb[pl.ds(3 * C + off, 16)] = (h3 & 3) * 2

        def hash_body(i, _):
            for u in range(UNROLL):
                hash_one(i * UNROLL + u)
            return 0

        lax.fori_loop(0, GH // UNROLL, hash_body, 0)

    def fire_level(b):
        return [pltpu.async_copy(
            tab_hbm.at[idxv[b].at[r]],
            rows[b].at[pl.ds(r * IDX_BLK, IDX_BLK)],
            sems[b]) for r in range(4 * RPC)]

    def bil_level(li, b):
        wxb, wyb, selb, rowsb = wxv[b], wyv[b], selv[b], rows[b]
        col0 = jnp.full((16,), 2 * li, jnp.int32)
        col1 = col0 + 1

        def bil_one(g):
            off = g * 16
            rowbase = iota + off
            wx = wxb[pl.ds(off, 16)]
            wy = wyb[pl.ds(off, 16)]
            s0 = selb[pl.ds(off, 16)]
            s1 = selb[pl.ds(C + off, 16)]
            s2 = selb[pl.ds(2 * C + off, 16)]
            s3 = selb[pl.ds(3 * C + off, 16)]
            omx = 1.0 - wx
            omy = 1.0 - wy
            for f, colv in ((0, col0), (1, col1)):
                r0 = plsc.load_gather(rowsb, [rowbase, s0 + f])
                r1 = plsc.load_gather(rowsb, [rowbase + C, s1 + f])
                r2 = plsc.load_gather(rowsb, [rowbase + 2 * C, s2 + f])
                r3 = plsc.load_gather(rowsb, [rowbase + 3 * C, s3 + f])
                c0 = r0 * omx + r2 * wx
                c1 = r1 * omx + r3 * wx
                plsc.store_scatter(outt, [rowbase, colv], c0 * omy + c1 * wy)

        def bil_body(i, _):
            for u in range(UNROLL):
                bil_one(i * UNROLL + u)
            return 0

        lax.fori_loop(0, GH // UNROLL, bil_body, 0)

    def chunk_body(ci, _):
        base = base0 + ci * C
        pltpu.sync_copy(xs_hbm.at[pl.ds(base, C)], xsv)
        pltpu.sync_copy(ys_hbm.at[pl.ds(base, C)], ysv)

        hash_level(0, 0)
        cps = fire_level(0)
        for li in range(N_LEVELS):
            b = li & 1
            nxt_cps = None
            if li + 1 < N_LEVELS:
                hash_level(li + 1, 1 - b)
                nxt_cps = fire_level(1 - b)
            for cp in cps:
                cp.wait()
            bil_level(li, b)
            cps = nxt_cps

        pltpu.sync_copy(outt, out_hbm.at[pl.ds(base, C)])
        return 0

    lax.fori_loop(0, NCHUNK, chunk_body, 0)


def kernel(x, tables):
    xs = x[:, 0]
    ys = x[:, 1]
    tab = tables.reshape(N_LEVELS * TBL * N_FEAT // PACK, PACK)
    mesh = plsc.VectorSubcoreMesh(
        core_axis_name="c", subcore_axis_name="s",
        num_cores=NC, num_subcores=NS)
    f = pl.kernel(
        _body,
        out_type=jax.ShapeDtypeStruct((N_PTS, N_LEVELS * N_FEAT), jnp.float32),
        mesh=mesh,
        compiler_params=pltpu.CompilerParams(
            needs_layout_passes=False, use_tc_tiling_on_sc=False),
        scratch_types=[
            pltpu.VMEM((C,), jnp.float32),
            pltpu.VMEM((C,), jnp.float32),
            pltpu.VMEM((C,), jnp.float32),
            pltpu.VMEM((C,), jnp.float32),
            pltpu.VMEM((C,), jnp.float32),
            pltpu.VMEM((C,), jnp.float32),
            pltpu.VMEM((4 * RPC, IDX_BLK), jnp.int32),
            pltpu.VMEM((4 * C,), jnp.int32),
            pltpu.VMEM((4 * RPC, IDX_BLK), jnp.int32),
            pltpu.VMEM((4 * C,), jnp.int32),
            pltpu.VMEM((4 * C, PACK), jnp.float32),
            pltpu.VMEM((4 * C, PACK), jnp.float32),
            pltpu.VMEM((C, N_LEVELS * N_FEAT), jnp.float32),
            pltpu.SemaphoreType.DMA,
            pltpu.SemaphoreType.DMA,
        ],
    )
    return f(xs, ys, tab)
